# Initial kernel scaffold; baseline (speedup 1.0000x reference)
import functools

import jax
import jax.numpy as jnp
from jax import lax
from jax.experimental import pallas as pl
from jax.experimental.pallas import tpu as pltpu
from jax.experimental.pallas import tpu_sc as plsc

F32 = jnp.float32


def _precompute_body(h, xpad, w1a, w1b, be1, outa, outb):
    ha = jnp.dot(h[...], w1a[...], preferred_element_type=F32) + be1[...]
    hb = jnp.dot(h[...], w1b[...], preferred_element_type=F32)
    outa[:, 0:128] = ha
    outa[:, 128:144] = xpad[...]
    outb[:, 0:128] = hb
    outb[:, 128:144] = xpad[...]


def _edge_body(ga, gb, ea, w1e, w1r, w2, b2, wat, ba, out):
    a = ga[...]
    b = gb[...]
    d = a[:, 128:144] - b[:, 128:144]
    r2 = jnp.sum(d * d, axis=1, keepdims=True)
    radial = jnp.sqrt(r2)
    z = (a[:, 0:128] + b[:, 0:128] + radial * w1r[...]
         + jnp.dot(ea[...], w1e[...], preferred_element_type=F32))
    m = z * jax.nn.sigmoid(z)
    y = jnp.dot(m, w2[...], preferred_element_type=F32) + b2[...]
    m2 = y * jax.nn.sigmoid(y)
    att_logit = jnp.sum(m2 * wat[...], axis=1, keepdims=True) + ba[...]
    out[...] = m2 * jax.nn.sigmoid(att_logit)


def _node_body(h, s0, s1, wh1a, wh1b, bh1, wh2, bh2, out):
    hv = h[...]
    s = s0[...] + s1[...]
    t = (jnp.dot(hv, wh1a[...], preferred_element_type=F32)
         + jnp.dot(s, wh1b[...], preferred_element_type=F32) + bh1[...])
    t = t * jax.nn.sigmoid(t)
    out[...] = hv + jnp.dot(t, wh2[...], preferred_element_type=F32) + bh2[...]


def _make_gather(n, e, cols):
    info = plsc.get_sparse_core_info()
    nc, ns = info.num_cores, info.num_subcores
    nw = nc * ns
    epw = e // nw
    chunk = 80
    nchunk = epw // chunk
    mesh = plsc.VectorSubcoreMesh(core_axis_name="c", subcore_axis_name="s")

    @functools.partial(
        pl.kernel, mesh=mesh,
        out_type=[jax.ShapeDtypeStruct((e, cols), F32),
                  jax.ShapeDtypeStruct((e, cols), F32)],
        scratch_types=[pltpu.VMEM((chunk,), jnp.int32),
                       pltpu.VMEM((chunk,), jnp.int32),
                       pltpu.VMEM((chunk, cols), F32),
                       pltpu.VMEM((chunk, cols), F32),
                       pltpu.SemaphoreType.DMA,
                       pltpu.SemaphoreType.DMA],
    )
    def gather_k(ta, tb, row, col, outa, outb, idxr, idxc, bufa, bufb,
                 sema, semb):
        wid = lax.axis_index("s") * nc + lax.axis_index("c")
        base = wid * epw

        def body(i, carry):
            cb = pl.multiple_of(base + i * chunk, 8)
            pltpu.sync_copy(row.at[pl.ds(cb, chunk)], idxr)
            pltpu.sync_copy(col.at[pl.ds(cb, chunk)], idxc)
            ca = pltpu.async_copy(ta.at[idxr], bufa, sema)
            cb2 = pltpu.async_copy(tb.at[idxc], bufb, semb)
            ca.wait()
            cb2.wait()
            pltpu.sync_copy(bufa, outa.at[pl.ds(cb, chunk)])
            pltpu.sync_copy(bufb, outb.at[pl.ds(cb, chunk)])
            return carry

        lax.fori_loop(0, nchunk, body, 0)

    return gather_k


def _make_scatter(n, e, d):
    info = plsc.get_sparse_core_info()
    nc, ns = info.num_cores, info.num_subcores
    nw = nc * ns
    epw = e // nw
    chunk = 80
    nchunk = epw // chunk
    rps = n // ns
    mesh = plsc.VectorSubcoreMesh(core_axis_name="c", subcore_axis_name="s")

    @functools.partial(
        pl.kernel, mesh=mesh,
        out_type=jax.ShapeDtypeStruct((nc * n, d), F32),
        scratch_types=[pltpu.VMEM((chunk,), jnp.int32),
                       pltpu.VMEM((chunk, d), F32),
                       pltpu.VMEM_SHARED((n, d), F32)],
    )
    def scatter_k(msg, row, zeros, out, idxv, mbuf, acc):
        c = lax.axis_index("c")
        s = lax.axis_index("s")
        wid = s * nc + c
        pltpu.sync_copy(zeros.at[pl.ds(s * rps, rps)],
                        acc.at[pl.ds(s * rps, rps)])
        plsc.subcore_barrier()
        base = wid * epw

        def body(i, carry):
            cb = pl.multiple_of(base + i * chunk, 8)
            pltpu.sync_copy(row.at[pl.ds(cb, chunk)], idxv)
            pltpu.sync_copy(msg.at[pl.ds(cb, chunk)], mbuf)
            pltpu.sync_copy(mbuf, acc.at[idxv], add=True)
            return carry

        lax.fori_loop(0, nchunk, body, 0)
        plsc.subcore_barrier()
        pltpu.sync_copy(acc.at[pl.ds(s * rps, rps)],
                        out.at[pl.ds(c * n + s * rps, rps)])

    return scatter_k


def kernel(x, h, edge_index, edge_attr, We1, be1, We2, be2, Wa, ba,
           Wh1, bh1, Wh2, bh2):
    n, d = h.shape
    e = edge_attr.shape[0]
    de = edge_attr.shape[1]
    cols = 144

    row = edge_index[0].astype(jnp.int32)
    col = edge_index[1].astype(jnp.int32)
    xpad = jnp.pad(x.astype(F32), ((0, 0), (0, 16 - x.shape[1])))

    w1a = We1[:d]
    w1b = We1[d:2 * d]
    w1r = We1[2 * d:2 * d + 1]
    w1e = We1[2 * d + 1:]

    nb = 2000
    grid_n = n // nb
    full = lambda shape: pl.BlockSpec(shape, lambda i: (0, 0))
    rowblk = lambda r, c_: pl.BlockSpec((r, c_), lambda i: (i, 0))

    ta, tb = pl.pallas_call(
        _precompute_body,
        grid=(grid_n,),
        in_specs=[rowblk(nb, d), rowblk(nb, 16), full((d, 128)),
                  full((d, 128)), full((1, 128))],
        out_specs=[rowblk(nb, cols), rowblk(nb, cols)],
        out_shape=[jax.ShapeDtypeStruct((n, cols), F32),
                   jax.ShapeDtypeStruct((n, cols), F32)],
    )(h, xpad, w1a, w1b, be1.reshape(1, 128))

    ga, gb = _make_gather(n, e, cols)(ta, tb, row, col)

    eb = 2560
    msg = pl.pallas_call(
        _edge_body,
        grid=(e // eb,),
        in_specs=[rowblk(eb, cols), rowblk(eb, cols), rowblk(eb, de),
                  full((de, 128)), full((1, 128)), full((128, 128)),
                  full((1, 128)), full((1, 128)), full((1, 1))],
        out_specs=rowblk(eb, 128),
        out_shape=jax.ShapeDtypeStruct((e, 128), F32),
    )(ga, gb, edge_attr, w1e, w1r, We2, be2.reshape(1, 128),
      Wa.reshape(1, 128), ba.reshape(1, 1))

    partials = _make_scatter(n, e, 128)(msg, row, jnp.zeros((n, 128), F32))
    s0 = partials[:n]
    s1 = partials[n:]

    out = pl.pallas_call(
        _node_body,
        grid=(grid_n,),
        in_specs=[rowblk(nb, d), rowblk(nb, 128), rowblk(nb, 128),
                  full((128, 128)), full((128, 128)), full((1, 128)),
                  full((128, 128)), full((1, 128))],
        out_specs=rowblk(nb, d),
        out_shape=jax.ShapeDtypeStruct((n, d), F32),
    )(h, s0, s1, Wh1[:d], Wh1[d:], bh1.reshape(1, 128), Wh2,
      bh2.reshape(1, 128))

    return out


# speedup vs baseline: 4.0901x; 4.0901x over previous
import functools

import jax
import jax.numpy as jnp
from jax import lax
from jax.experimental import pallas as pl
from jax.experimental.pallas import tpu as pltpu
from jax.experimental.pallas import tpu_sc as plsc

F32 = jnp.float32


def _precompute_body(h, w1a, w1b, be1, outa, outb):
    hv = h[...]
    outa[...] = jnp.dot(hv, w1a[...], preferred_element_type=F32) + be1[...]
    outb[...] = jnp.dot(hv, w1b[...], preferred_element_type=F32)


def _edge_body(ga, gb, r2, ea, w1e, w1r, w2, b2, wat, ba, out):
    radial = jnp.sqrt(r2[...])
    z = (ga[...] + gb[...] + radial * w1r[...]
         + jnp.dot(ea[...], w1e[...], preferred_element_type=F32))
    m = z * jax.nn.sigmoid(z)
    y = jnp.dot(m, w2[...], preferred_element_type=F32) + b2[...]
    m2 = y * jax.nn.sigmoid(y)
    att_logit = jnp.sum(m2 * wat[...], axis=1, keepdims=True) + ba[...]
    out[...] = m2 * jax.nn.sigmoid(att_logit)


def _node_body(h, s0, s1, wh1a, wh1b, bh1, wh2, bh2, out):
    hv = h[...]
    s = s0[...] + s1[...]
    t = (jnp.dot(hv, wh1a[...], preferred_element_type=F32)
         + jnp.dot(s, wh1b[...], preferred_element_type=F32) + bh1[...])
    t = t * jax.nn.sigmoid(t)
    out[...] = hv + jnp.dot(t, wh2[...], preferred_element_type=F32) + bh2[...]


def _make_gather(n, e, d):
    info = plsc.get_sparse_core_info()
    nc, ns, nl = info.num_cores, info.num_subcores, info.num_lanes
    nw = nc * ns
    epw = e // nw
    chunk = 80
    nchunk = epw // chunk
    groups = chunk // nl
    mesh = plsc.VectorSubcoreMesh(core_axis_name="c", subcore_axis_name="s")

    @functools.partial(
        pl.kernel, mesh=mesh,
        out_type=[jax.ShapeDtypeStruct((e, d), F32),
                  jax.ShapeDtypeStruct((e, d), F32),
                  jax.ShapeDtypeStruct((e,), F32)],
        scratch_types=[pltpu.VMEM((chunk,), jnp.int32),
                       pltpu.VMEM((chunk,), jnp.int32),
                       pltpu.VMEM((chunk, d), F32),
                       pltpu.VMEM((chunk, d), F32),
                       pltpu.VMEM((chunk,), F32),
                       pltpu.VMEM((n,), F32),
                       pltpu.VMEM((n,), F32),
                       pltpu.VMEM((n,), F32),
                       pltpu.SemaphoreType.DMA,
                       pltpu.SemaphoreType.DMA],
        compiler_params=pltpu.CompilerParams(needs_layout_passes=False),
    )
    def gather_k(ta, tb, x0, x1, x2, row, col, outa, outb, outr2,
                 idxr, idxc, bufa, bufb, r2buf, xa, xb, xc, sema, semb):
        wid = lax.axis_index("s") * nc + lax.axis_index("c")
        base = wid * epw
        pltpu.sync_copy(x0, xa)
        pltpu.sync_copy(x1, xb)
        pltpu.sync_copy(x2, xc)

        def body(i, carry):
            cb = pl.multiple_of(base + i * chunk, 8)
            pltpu.sync_copy(row.at[pl.ds(cb, chunk)], idxr)
            pltpu.sync_copy(col.at[pl.ds(cb, chunk)], idxc)
            ca = pltpu.async_copy(ta.at[idxr], bufa, sema)
            cb2 = pltpu.async_copy(tb.at[idxc], bufb, semb)
            for g in range(groups):
                ir = idxr[pl.ds(g * nl, nl)]
                ic = idxc[pl.ds(g * nl, nl)]
                dx = plsc.load_gather(xa, [ir]) - plsc.load_gather(xa, [ic])
                dy = plsc.load_gather(xb, [ir]) - plsc.load_gather(xb, [ic])
                dz = plsc.load_gather(xc, [ir]) - plsc.load_gather(xc, [ic])
                r2buf[pl.ds(g * nl, nl)] = dx * dx + dy * dy + dz * dz
            ca.wait()
            cb2.wait()
            pltpu.sync_copy(bufa, outa.at[pl.ds(cb, chunk)])
            pltpu.sync_copy(bufb, outb.at[pl.ds(cb, chunk)])
            pltpu.sync_copy(r2buf, outr2.at[pl.ds(cb, chunk)])
            return carry

        lax.fori_loop(0, nchunk, body, 0)

    return gather_k


def _make_scatter(n, e, d):
    info = plsc.get_sparse_core_info()
    nc, ns = info.num_cores, info.num_subcores
    nw = nc * ns
    epw = e // nw
    chunk = 80
    nchunk = epw // chunk
    rps = -(-n // (8 * ns)) * 8
    npad = rps * ns
    mesh = plsc.VectorSubcoreMesh(core_axis_name="c", subcore_axis_name="s")

    @functools.partial(
        pl.kernel, mesh=mesh,
        out_type=jax.ShapeDtypeStruct((nc * npad, d), F32),
        scratch_types=[pltpu.VMEM((chunk,), jnp.int32),
                       pltpu.VMEM((chunk, d), F32),
                       pltpu.VMEM_SHARED((npad, d), F32)],
    )
    def scatter_k(msg, row, zeros, out, idxv, mbuf, acc):
        c = lax.axis_index("c")
        s = lax.axis_index("s")
        wid = s * nc + c
        pltpu.sync_copy(zeros.at[pl.ds(pl.multiple_of(s * rps, 8), rps)],
                        acc.at[pl.ds(pl.multiple_of(s * rps, 8), rps)])
        plsc.subcore_barrier()
        base = wid * epw

        def body(i, carry):
            cb = pl.multiple_of(base + i * chunk, 8)
            pltpu.sync_copy(row.at[pl.ds(cb, chunk)], idxv)
            pltpu.sync_copy(msg.at[pl.ds(cb, chunk)], mbuf)
            pltpu.sync_copy(mbuf, acc.at[idxv], add=True)
            return carry

        lax.fori_loop(0, nchunk, body, 0)
        plsc.subcore_barrier()
        pltpu.sync_copy(acc.at[pl.ds(pl.multiple_of(s * rps, 8), rps)],
                        out.at[pl.ds(pl.multiple_of(c * npad + s * rps, 8),
                                     rps)])

    return scatter_k, npad


def kernel(x, h, edge_index, edge_attr, We1, be1, We2, be2, Wa, ba,
           Wh1, bh1, Wh2, bh2):
    n, d = h.shape
    e = edge_attr.shape[0]
    de = edge_attr.shape[1]

    row = edge_index[0].astype(jnp.int32)
    col = edge_index[1].astype(jnp.int32)
    xf = x.astype(F32)

    w1a = We1[:d]
    w1b = We1[d:2 * d]
    w1r = We1[2 * d:2 * d + 1]
    w1e = We1[2 * d + 1:]

    nb = 2000
    grid_n = n // nb
    full = lambda shape: pl.BlockSpec(shape, lambda i: tuple(0 for _ in shape))
    rowblk = lambda r, c_: pl.BlockSpec((r, c_), lambda i: (i, 0))

    ta, tb = pl.pallas_call(
        _precompute_body,
        grid=(grid_n,),
        in_specs=[rowblk(nb, d), full((d, 128)), full((d, 128)),
                  full((1, 128))],
        out_specs=[rowblk(nb, 128), rowblk(nb, 128)],
        out_shape=[jax.ShapeDtypeStruct((n, 128), F32),
                   jax.ShapeDtypeStruct((n, 128), F32)],
    )(h, w1a, w1b, be1.reshape(1, 128))

    ga, gb, r2 = _make_gather(n, e, 128)(
        ta, tb, xf[:, 0], xf[:, 1], xf[:, 2], row, col)

    eb = 2560
    msg = pl.pallas_call(
        _edge_body,
        grid=(e // eb,),
        in_specs=[rowblk(eb, 128), rowblk(eb, 128), rowblk(eb, 1),
                  rowblk(eb, de), full((de, 128)), full((1, 128)),
                  full((128, 128)), full((1, 128)), full((1, 128)),
                  full((1, 1))],
        out_specs=rowblk(eb, 128),
        out_shape=jax.ShapeDtypeStruct((e, 128), F32),
    )(ga, gb, r2.reshape(e, 1), edge_attr, w1e, w1r, We2,
      be2.reshape(1, 128), Wa.reshape(1, 128), ba.reshape(1, 1))

    scatter_k, npad = _make_scatter(n, e, 128)
    partials = scatter_k(msg, row, jnp.zeros((npad, 128), F32))
    s0 = partials[:n]
    s1 = partials[npad:npad + n]

    out = pl.pallas_call(
        _node_body,
        grid=(grid_n,),
        in_specs=[rowblk(nb, d), rowblk(nb, 128), rowblk(nb, 128),
                  full((128, 128)), full((128, 128)), full((1, 128)),
                  full((128, 128)), full((1, 128))],
        out_specs=rowblk(nb, d),
        out_shape=jax.ShapeDtypeStruct((n, d), F32),
    )(h, s0, s1, Wh1[:d], Wh1[d:], bh1.reshape(1, 128), Wh2,
      bh2.reshape(1, 128))

    return out
